# baseline (speedup 1.0000x reference)
import jax
import jax.numpy as jnp
from jax.experimental import pallas as pl
from jax.experimental.pallas import tpu as pltpu

_D = 64
_K = 1024
_NCB = 4
_ROWS = 1024
_CHUNKS = 2


def _split3(cb):
    b1 = cb.astype(jnp.bfloat16)
    r1 = cb - b1.astype(jnp.float32)
    b2 = r1.astype(jnp.bfloat16)
    r2 = r1 - b2.astype(jnp.float32)
    b3 = r2.astype(jnp.bfloat16)
    return b1, b2, b3


def _rvq_body(x_ref, win_ref, bin_ref, wout_ref, bout_ref, csq_ref,
              ct0_ref, ct1_ref, ct2_ref, ct3_ref,
              *rest):
    g_refs = rest[:12]
    recon_ref, idx_ref = rest[12], rest[13]
    x = x_ref[...]
    z = jnp.dot(x.astype(jnp.bfloat16), win_ref[...],
                preferred_element_type=jnp.float32) + bin_ref[...]
    cts = (ct0_ref, ct1_ref, ct2_ref, ct3_ref)
    half = _ROWS // _CHUNKS
    lanes = jax.lax.broadcasted_iota(jnp.int32, (half, _K), 1)
    recon_parts, idx_parts = [], []
    for c in range(_CHUNKS):
        residual = z[c * half:(c + 1) * half, :]
        z_q = jnp.zeros_like(residual)
        idx_cols = []
        for k in range(_NCB):
            ct = cts[k][...]
            csq = csq_ref[k:k + 1, :]
            rsq = jnp.sum(residual * residual, axis=1, keepdims=True)
            scores = jnp.dot(residual.astype(jnp.bfloat16),
                             ct.astype(jnp.bfloat16),
                             preferred_element_type=jnp.float32)
            dist = rsq - 2.0 * scores + csq
            m = jnp.min(dist, axis=1, keepdims=True)
            ind = jnp.min(jnp.where(dist == m, lanes, _K), axis=1,
                          keepdims=True)
            onehot = (lanes == ind).astype(jnp.bfloat16)
            q1 = jnp.dot(onehot, g_refs[3 * k][...],
                         preferred_element_type=jnp.float32)
            q2 = jnp.dot(onehot, g_refs[3 * k + 1][...],
                         preferred_element_type=jnp.float32)
            q3 = jnp.dot(onehot, g_refs[3 * k + 2][...],
                         preferred_element_type=jnp.float32)
            q = (q1 + q2) + q3
            residual = residual - q
            z_q = z_q + q
            idx_cols.append(ind)
        recon_parts.append(
            jnp.dot(z_q.astype(jnp.bfloat16), wout_ref[...],
                    preferred_element_type=jnp.float32) + bout_ref[...])
        idx_parts.append(jnp.concatenate(idx_cols, axis=1))
    recon_ref[...] = jnp.concatenate(recon_parts, axis=0)
    idx_ref[...] = jnp.concatenate(idx_parts, axis=0)


def kernel(mel_frame, W_in, b_in, W_out, b_out, cb0, cb1, cb2, cb3):
    Bb, Tt, Mm = mel_frame.shape
    N = Bb * Tt
    x = mel_frame.reshape(N, Mm)

    def full(shape):
        return pl.BlockSpec(shape, lambda i: (0, 0))

    splits = []
    for cb in (cb0, cb1, cb2, cb3):
        splits.extend(_split3(cb))
    csq = jnp.stack([jnp.sum(cb * cb, axis=-1)
                     for cb in (cb0, cb1, cb2, cb3)])

    recon, inds = pl.pallas_call(
        _rvq_body,
        grid=(N // _ROWS,),
        in_specs=[
            pl.BlockSpec((_ROWS, Mm), lambda i: (i, 0)),
            full((Mm, _D)), full((1, _D)), full((_D, Mm)), full((1, Mm)),
            full((_NCB, _K)),
            full((_D, _K)), full((_D, _K)), full((_D, _K)), full((_D, _K)),
        ] + [full((_K, _D))] * 12,
        out_specs=[
            pl.BlockSpec((_ROWS, Mm), lambda i: (i, 0)),
            pl.BlockSpec((_ROWS, _NCB), lambda i: (i, 0)),
        ],
        out_shape=[
            jax.ShapeDtypeStruct((N, Mm), jnp.float32),
            jax.ShapeDtypeStruct((N, _NCB), jnp.int32),
        ],
        compiler_params=pltpu.CompilerParams(
            dimension_semantics=("arbitrary",)),
    )(x, W_in.astype(jnp.bfloat16), b_in.reshape(1, _D),
      W_out.astype(jnp.bfloat16), b_out.reshape(1, Mm), csq,
      cb0.T, cb1.T, cb2.T, cb3.T, *splits)
    return recon.reshape(Bb, Tt, Mm), inds.reshape(Bb, Tt, _NCB)


# speedup vs baseline: 1.7553x; 1.7553x over previous
import jax
import jax.numpy as jnp
from jax.experimental import pallas as pl
from jax.experimental.pallas import tpu as pltpu

_D = 64
_K = 1024
_NCB = 4
_ROWS = 1024
_CHUNKS = 2


def _split3(cb):
    b1 = cb.astype(jnp.bfloat16)
    r1 = cb - b1.astype(jnp.float32)
    b2 = r1.astype(jnp.bfloat16)
    r2 = r1 - b2.astype(jnp.float32)
    b3 = r2.astype(jnp.bfloat16)
    return b1, b2, b3


def _rvq_body(x_ref, win_ref, bin_ref, wout_ref, bout_ref, csq_ref,
              ct0_ref, ct1_ref, ct2_ref, ct3_ref,
              *rest):
    g_refs = rest[:4]
    recon_ref, idx_ref = rest[4], rest[5]
    x = x_ref[...]
    z = jnp.dot(x.astype(jnp.bfloat16), win_ref[...],
                preferred_element_type=jnp.float32) + bin_ref[...]
    cts = (ct0_ref, ct1_ref, ct2_ref, ct3_ref)
    lanes = jax.lax.broadcasted_iota(jnp.int32, (_ROWS, _K), 1)
    residual = z
    z_q = jnp.zeros_like(z)
    for k in range(_NCB):
        ct = cts[k][...]
        csq = csq_ref[k:k + 1, :]
        rsq = jnp.sum(residual * residual, axis=1, keepdims=True)
        scores = jnp.dot(residual.astype(jnp.bfloat16),
                         ct.astype(jnp.bfloat16),
                         preferred_element_type=jnp.float32)
        dist = rsq - 2.0 * scores + csq
        m = jnp.min(dist, axis=1, keepdims=True)
        ind = jnp.min(jnp.where(dist == m, lanes, _K), axis=1,
                      keepdims=True)
        onehot = (lanes == ind).astype(jnp.bfloat16)
        qcat = jnp.dot(onehot, g_refs[k][...],
                       preferred_element_type=jnp.float32)
        q = (qcat[:, :_D] + qcat[:, _D:2 * _D]) + qcat[:, 2 * _D:3 * _D]
        residual = residual - q
        z_q = z_q + q
        idx_ref[:, k:k + 1] = ind
    recon_ref[...] = (
        jnp.dot(z_q.astype(jnp.bfloat16), wout_ref[...],
                preferred_element_type=jnp.float32)
        + bout_ref[...])


def kernel(mel_frame, W_in, b_in, W_out, b_out, cb0, cb1, cb2, cb3):
    Bb, Tt, Mm = mel_frame.shape
    N = Bb * Tt
    x = mel_frame.reshape(N, Mm)

    def full(shape):
        return pl.BlockSpec(shape, lambda i: (0, 0))

    splits = [jnp.concatenate(_split3(cb), axis=1)
              for cb in (cb0, cb1, cb2, cb3)]
    csq = jnp.stack([jnp.sum(cb * cb, axis=-1)
                     for cb in (cb0, cb1, cb2, cb3)])

    recon, inds = pl.pallas_call(
        _rvq_body,
        grid=(N // _ROWS,),
        in_specs=[
            pl.BlockSpec((_ROWS, Mm), lambda i: (i, 0)),
            full((Mm, _D)), full((1, _D)), full((_D, Mm)), full((1, Mm)),
            full((_NCB, _K)),
            full((_D, _K)), full((_D, _K)), full((_D, _K)), full((_D, _K)),
        ] + [full((_K, 3 * _D))] * 4,
        out_specs=[
            pl.BlockSpec((_ROWS, Mm), lambda i: (i, 0)),
            pl.BlockSpec((_ROWS, _NCB), lambda i: (i, 0)),
        ],
        out_shape=[
            jax.ShapeDtypeStruct((N, Mm), jnp.float32),
            jax.ShapeDtypeStruct((N, _NCB), jnp.int32),
        ],
        compiler_params=pltpu.CompilerParams(
            dimension_semantics=("arbitrary",)),
    )(x, W_in.astype(jnp.bfloat16), b_in.reshape(1, _D),
      W_out.astype(jnp.bfloat16), b_out.reshape(1, Mm), csq,
      cb0.T, cb1.T, cb2.T, cb3.T, *splits)
    return recon.reshape(Bb, Tt, Mm), inds.reshape(Bb, Tt, _NCB)


# speedup vs baseline: 1.7581x; 1.0016x over previous
import jax
import jax.numpy as jnp
from jax.experimental import pallas as pl
from jax.experimental.pallas import tpu as pltpu

_D = 64
_K = 1024
_NCB = 4
_ROWS = 1024
_CHUNKS = 2


def _split3(cb):
    b1 = cb.astype(jnp.bfloat16)
    r1 = cb - b1.astype(jnp.float32)
    b2 = r1.astype(jnp.bfloat16)
    r2 = r1 - b2.astype(jnp.float32)
    b3 = r2.astype(jnp.bfloat16)
    return b1, b2, b3


def _rvq_body(x_ref, win_ref, bin_ref, wout_ref, bout_ref, csq_ref,
              ct0_ref, ct1_ref, ct2_ref, ct3_ref,
              *rest):
    g_refs = rest[:4]
    recon_ref, idx_ref = rest[4], rest[5]
    x = x_ref[...]
    z = jnp.dot(x.astype(jnp.bfloat16), win_ref[...],
                preferred_element_type=jnp.float32) + bin_ref[...]
    cts = (ct0_ref, ct1_ref, ct2_ref, ct3_ref)
    lanes = jax.lax.broadcasted_iota(jnp.int32, (_ROWS, _K), 1)
    residual = z
    z_q = jnp.zeros_like(z)
    for k in range(_NCB):
        rsq = jnp.sum(residual * residual, axis=1, keepdims=True)
        key = (rsq + jnp.dot(residual.astype(jnp.bfloat16), cts[k][...],
                             preferred_element_type=jnp.float32)
               ) + csq_ref[k:k + 1, :]
        m = jnp.min(key, axis=1, keepdims=True)
        ind = jnp.min(jnp.where(key == m, lanes, _K), axis=1,
                      keepdims=True)
        onehot = (lanes == ind).astype(jnp.bfloat16)
        qcat = jnp.dot(onehot, g_refs[k][...],
                       preferred_element_type=jnp.float32)
        q = (qcat[:, :_D] + qcat[:, _D:2 * _D]) + qcat[:, 2 * _D:3 * _D]
        residual = residual - q
        z_q = z_q + q
        idx_ref[:, k:k + 1] = ind
    recon_ref[...] = (
        jnp.dot(z_q.astype(jnp.bfloat16), wout_ref[...],
                preferred_element_type=jnp.float32)
        + bout_ref[...])


def kernel(mel_frame, W_in, b_in, W_out, b_out, cb0, cb1, cb2, cb3):
    Bb, Tt, Mm = mel_frame.shape
    N = Bb * Tt
    x = mel_frame.reshape(N, Mm)

    def full(shape):
        return pl.BlockSpec(shape, lambda i: (0, 0))

    splits = [jnp.concatenate(_split3(cb), axis=1)
              for cb in (cb0, cb1, cb2, cb3)]
    csq = jnp.stack([jnp.sum(cb * cb, axis=-1)
                     for cb in (cb0, cb1, cb2, cb3)])

    recon, inds = pl.pallas_call(
        _rvq_body,
        grid=(N // _ROWS,),
        in_specs=[
            pl.BlockSpec((_ROWS, Mm), lambda i: (i, 0)),
            full((Mm, _D)), full((1, _D)), full((_D, Mm)), full((1, Mm)),
            full((_NCB, _K)),
            full((_D, _K)), full((_D, _K)), full((_D, _K)), full((_D, _K)),
        ] + [full((_K, 3 * _D))] * 4,
        out_specs=[
            pl.BlockSpec((_ROWS, Mm), lambda i: (i, 0)),
            pl.BlockSpec((_ROWS, _NCB), lambda i: (i, 0)),
        ],
        out_shape=[
            jax.ShapeDtypeStruct((N, Mm), jnp.float32),
            jax.ShapeDtypeStruct((N, _NCB), jnp.int32),
        ],
        compiler_params=pltpu.CompilerParams(
            dimension_semantics=("parallel",)),
    )(x, W_in.astype(jnp.bfloat16), b_in.reshape(1, _D),
      W_out.astype(jnp.bfloat16), b_out.reshape(1, Mm), csq,
      (-2.0 * cb0.T).astype(jnp.bfloat16), (-2.0 * cb1.T).astype(jnp.bfloat16),
      (-2.0 * cb2.T).astype(jnp.bfloat16), (-2.0 * cb3.T).astype(jnp.bfloat16),
      *splits)
    return recon.reshape(Bb, Tt, Mm), inds.reshape(Bb, Tt, _NCB)


# speedup vs baseline: 1.9129x; 1.0880x over previous
import jax
import jax.numpy as jnp
from jax.experimental import pallas as pl
from jax.experimental.pallas import tpu as pltpu

_D = 64
_K = 1024
_NCB = 4
_ROWS = 2048
_CHUNKS = 2


def _split3(cb):
    b1 = cb.astype(jnp.bfloat16)
    r1 = cb - b1.astype(jnp.float32)
    b2 = r1.astype(jnp.bfloat16)
    r2 = r1 - b2.astype(jnp.float32)
    b3 = r2.astype(jnp.bfloat16)
    return b1, b2, b3


def _rvq_body(x_ref, win_ref, bin_ref, wout_ref, bout_ref, csq_ref,
              ct0_ref, ct1_ref, ct2_ref, ct3_ref,
              *rest):
    g_refs = rest[:4]
    recon_ref, idx_ref = rest[4], rest[5]
    x = x_ref[...]
    z = jnp.dot(x.astype(jnp.bfloat16), win_ref[...],
                preferred_element_type=jnp.float32) + bin_ref[...]
    cts = (ct0_ref, ct1_ref, ct2_ref, ct3_ref)
    lanes = jax.lax.broadcasted_iota(jnp.int32, (_ROWS, _K), 1)
    residual = z
    z_q = jnp.zeros_like(z)
    for k in range(_NCB):
        rsq = jnp.sum(residual * residual, axis=1, keepdims=True)
        key = (rsq + jnp.dot(residual.astype(jnp.bfloat16), cts[k][...],
                             preferred_element_type=jnp.float32)
               ) + csq_ref[k:k + 1, :]
        m = jnp.min(key, axis=1, keepdims=True)
        ind = jnp.min(jnp.where(key == m, lanes, _K), axis=1,
                      keepdims=True)
        onehot = (lanes == ind).astype(jnp.bfloat16)
        qcat = jnp.dot(onehot, g_refs[k][...],
                       preferred_element_type=jnp.float32)
        q = (qcat[:, :_D] + qcat[:, _D:2 * _D]) + qcat[:, 2 * _D:3 * _D]
        residual = residual - q
        z_q = z_q + q
        idx_ref[:, k:k + 1] = ind
    recon_ref[...] = (
        jnp.dot(z_q.astype(jnp.bfloat16), wout_ref[...],
                preferred_element_type=jnp.float32)
        + bout_ref[...])


def kernel(mel_frame, W_in, b_in, W_out, b_out, cb0, cb1, cb2, cb3):
    Bb, Tt, Mm = mel_frame.shape
    N = Bb * Tt
    x = mel_frame.reshape(N, Mm)

    def full(shape):
        return pl.BlockSpec(shape, lambda i: (0, 0))

    splits = [jnp.concatenate(_split3(cb), axis=1)
              for cb in (cb0, cb1, cb2, cb3)]
    csq = jnp.stack([jnp.sum(cb * cb, axis=-1)
                     for cb in (cb0, cb1, cb2, cb3)])

    recon, inds = pl.pallas_call(
        _rvq_body,
        grid=(N // _ROWS,),
        in_specs=[
            pl.BlockSpec((_ROWS, Mm), lambda i: (i, 0)),
            full((Mm, _D)), full((1, _D)), full((_D, Mm)), full((1, Mm)),
            full((_NCB, _K)),
            full((_D, _K)), full((_D, _K)), full((_D, _K)), full((_D, _K)),
        ] + [full((_K, 3 * _D))] * 4,
        out_specs=[
            pl.BlockSpec((_ROWS, Mm), lambda i: (i, 0)),
            pl.BlockSpec((_ROWS, _NCB), lambda i: (i, 0)),
        ],
        out_shape=[
            jax.ShapeDtypeStruct((N, Mm), jnp.float32),
            jax.ShapeDtypeStruct((N, _NCB), jnp.int32),
        ],
        compiler_params=pltpu.CompilerParams(
            dimension_semantics=("parallel",)),
    )(x, W_in.astype(jnp.bfloat16), b_in.reshape(1, _D),
      W_out.astype(jnp.bfloat16), b_out.reshape(1, Mm), csq,
      (-2.0 * cb0.T).astype(jnp.bfloat16), (-2.0 * cb1.T).astype(jnp.bfloat16),
      (-2.0 * cb2.T).astype(jnp.bfloat16), (-2.0 * cb3.T).astype(jnp.bfloat16),
      *splits)
    return recon.reshape(Bb, Tt, Mm), inds.reshape(Bb, Tt, _NCB)
